# baseline (speedup 1.0000x reference)
import functools

import jax
import jax.numpy as jnp
import numpy as np
from jax import lax
from jax.experimental import pallas as pl
from jax.experimental.pallas import tpu as pltpu
from jax.experimental.pallas import tpu_sc as plsc

B, N = 128, 106
OUT_W = 512
UPSCALE = 4
STRIDE = 4
SIGMA = 2.0
RADIUS = int(SIGMA * 3)
KSIZE = 2 * RADIUS + 1

BL = 8
NH = N // 2
NL = 14
CHUNKS = (NH + NL - 1) // NL
NLAST = NH - NL * (CHUNKS - 1)

_GVALS = np.exp(-((np.arange(KSIZE) - RADIUS) ** 2.0) / (2.0 * SIGMA ** 2)).astype(np.float32)


def _sc_scatter_x(xpl, ypl):
    mesh = plsc.VectorSubcoreMesh(core_axis_name="c", subcore_axis_name="s")

    @functools.partial(
        pl.kernel,
        out_type=jax.ShapeDtypeStruct((N, B, OUT_W), jnp.float32),
        mesh=mesh,
        scratch_types=[
            pltpu.VMEM((2, NL, BL, OUT_W), jnp.float32),
            pltpu.VMEM((BL, N), jnp.float32),
            pltpu.VMEM((BL, N), jnp.float32),
            pltpu.SemaphoreType.DMA,
            pltpu.SemaphoreType.DMA,
        ],
        compiler_params=pltpu.CompilerParams(needs_layout_passes=False),
    )
    def k(x_hbm, y_hbm, out_x, buf, posv, othv, sem0, sem1):
        wid = lax.axis_index("s") * 2 + lax.axis_index("c")
        bg = lax.rem(wid, 16)
        b0 = bg * BL
        nbase = (wid // 16) * NH
        sems = (sem0, sem1)

        pltpu.sync_copy(x_hbm.at[pl.ds(b0, BL)], posv)
        pltpu.sync_copy(y_hbm.at[pl.ds(b0, BL)], othv)

        zeros16 = jnp.zeros((16,), jnp.float32)

        def zero_half(sl):
            def zbody(i, c):
                nl = i // BL
                bl = i - nl * BL
                for cc in range(OUT_W // 16):
                    buf[sl, nl, bl, pl.ds(cc * 16, 16)] = zeros16
                return c
            lax.fori_loop(0, NL * BL, zbody, 0)

        lanes = lax.iota(jnp.int32, 16)
        lane_hi = lanes >> 3
        lane_bl = lanes & 7
        gvecs = [jnp.full((16,), float(v), jnp.float32) for v in _GVALS]
        zvecs = [zeros16] * KSIZE

        def scatter_chunk(ci, slot, vals):
            nloc0 = ci * NL
            slotv = jnp.full((16,), slot, jnp.int32)

            def gbody(g, c):
                nlv = lane_hi + 2 * g
                nloc = nloc0 + nlv
                nv = nbase + nloc
                inb = nv < N
                p = plsc.load_gather(posv, [lane_bl, nv],
                                     mask=inb).astype(jnp.int32)
                o = plsc.load_gather(othv, [lane_bl, nv],
                                     mask=inb).astype(jnp.int32)
                ul = p - RADIUS
                br = p + RADIUS + 1
                ulo = o - RADIUS
                bro = o + RADIUS + 1
                in_ul = (ul >= 0) & (ul <= OUT_W) & (ulo >= 0) & (ulo <= OUT_W)
                in_br = (br >= 0) & (br <= OUT_W) & (bro >= 0) & (bro <= OUT_W)
                valid = (in_ul | in_br) & (nloc < NH)
                for j in range(KSIZE):
                    col = ul + j
                    m = valid & (col >= 0) & (col < OUT_W)
                    plsc.store_scatter(buf, [slotv, nlv, lane_bl, col],
                                       vals[j], mask=m)
                return c

            lax.fori_loop(0, NL * BL // 16, gbody, 0)

        def issue(ci, slot):
            n0 = nbase + ci * NL
            for sl in range(2):
                @pl.when((slot == sl) & (ci < CHUNKS - 1))
                def _d(sl=sl):
                    pltpu.async_copy(
                        buf.at[sl],
                        out_x.at[pl.ds(n0, NL), pl.ds(b0, BL)],
                        sems[sl])

            @pl.when(ci == CHUNKS - 1)
            def _dl():
                pltpu.async_copy(
                    buf.at[1, pl.ds(0, NLAST)],
                    out_x.at[pl.ds(n0, NLAST), pl.ds(b0, BL)],
                    sem1)

        zero_half(0)
        scatter_chunk(0, 0, gvecs)
        issue(0, 0)
        zero_half(1)

        def body(ci, c):
            slot = lax.rem(ci, 2)

            @pl.when(ci >= 2)
            def _drain_and_restore():
                for sl in range(2):
                    @pl.when(slot == sl)
                    def _w(sl=sl):
                        pltpu.make_async_copy(
                            buf.at[sl],
                            out_x.at[pl.ds(0, NL), pl.ds(0, BL)],
                            sems[sl]).wait()

                scatter_chunk(ci - 2, slot, zvecs)

            scatter_chunk(ci, slot, gvecs)
            issue(ci, slot)
            return c

        lax.fori_loop(1, CHUNKS, body, 0)

        pltpu.make_async_copy(
            buf.at[0], out_x.at[pl.ds(0, NL), pl.ds(0, BL)], sem0).wait()
        pltpu.make_async_copy(
            buf.at[1, pl.ds(0, NLAST)],
            out_x.at[pl.ds(0, NLAST), pl.ds(0, BL)], sem1).wait()

    return k(xpl, ypl)


def _tc_dense_y_body(x_ref, y_ref, o_ref):
    wio = lax.broadcasted_iota(jnp.int32, (8, OUT_W), 1).astype(jnp.float32)
    xall = x_ref[...].astype(jnp.int32)
    yall = y_ref[...].astype(jnp.int32)

    def nbody(nn, c):
        xc = pltpu.roll(xall, -nn, 1)[:, :1]
        yc = pltpu.roll(yall, -nn, 1)[:, :1]
        ulx = xc - RADIUS
        uly = yc - RADIUS
        brx = xc + RADIUS + 1
        bry = yc + RADIUS + 1
        in_ul = ((ulx >= 0) & (ulx <= OUT_W) & (uly >= 0) & (uly <= OUT_W))
        in_br = ((brx >= 0) & (brx <= OUT_W) & (bry >= 0) & (bry <= OUT_W))
        valid = in_ul | in_br
        center = jnp.where(valid, yc, -30000).astype(jnp.float32)
        d = wio - center
        t = d * d
        val = jnp.exp(t * (-1.0 / (2.0 * SIGMA ** 2)))
        o_ref[pl.ds(nn, 1)] = jnp.where(
            t <= float(RADIUS * RADIUS), val, 0.0).reshape(1, 8, OUT_W)
        return c

    lax.fori_loop(0, N, nbody, 0)


def _tc_dense_y(xpl, ypl, interpret=False):
    return pl.pallas_call(
        _tc_dense_y_body,
        out_shape=jax.ShapeDtypeStruct((N, B, OUT_W), jnp.float32),
        grid=(B // 8,),
        in_specs=[
            pl.BlockSpec((8, B), lambda j: (j, 0)),
            pl.BlockSpec((8, B), lambda j: (j, 0)),
        ],
        out_specs=pl.BlockSpec((N, 8, OUT_W), lambda j: (0, j, 0)),
        interpret=interpret,
    )(xpl, ypl)


def kernel(lmks):
    lm_scaled = lmks * UPSCALE / STRIDE
    xpl = lm_scaled[..., 0]
    ypl = lm_scaled[..., 1]
    xpad = jnp.pad(xpl, ((0, 0), (0, B - N)))
    ypad = jnp.pad(ypl, ((0, 0), (0, B - N)))
    ox = _sc_scatter_x(xpl, ypl)
    oy = _tc_dense_y(xpad, ypad)
    return ox.transpose(1, 0, 2), oy.transpose(1, 0, 2)


# speedup vs baseline: 7.2091x; 7.2091x over previous
import functools

import jax
import jax.numpy as jnp
import numpy as np
from jax import lax
from jax.experimental import pallas as pl
from jax.experimental.pallas import tpu as pltpu
from jax.experimental.pallas import tpu_sc as plsc

B, N = 128, 106
OUT_W = 512
UPSCALE = 4
STRIDE = 4
SIGMA = 2.0
RADIUS = int(SIGMA * 3)
KSIZE = 2 * RADIUS + 1

BL = 8
NH = N // 2
NL = 14
CHUNKS = (NH + NL - 1) // NL
NLAST = NH - NL * (CHUNKS - 1)

_GVALS = np.exp(-((np.arange(KSIZE) - RADIUS) ** 2.0) / (2.0 * SIGMA ** 2)).astype(np.float32)


def _sc_scatter_x(xpl, ypl):
    mesh = plsc.VectorSubcoreMesh(core_axis_name="c", subcore_axis_name="s")

    @functools.partial(
        pl.kernel,
        out_type=jax.ShapeDtypeStruct((N, B, OUT_W), jnp.float32),
        mesh=mesh,
        scratch_types=[
            pltpu.VMEM((2, NL, BL, OUT_W), jnp.float32),
            pltpu.VMEM((BL, N), jnp.float32),
            pltpu.VMEM((BL, N), jnp.float32),
            pltpu.SemaphoreType.DMA,
            pltpu.SemaphoreType.DMA,
        ],
        compiler_params=pltpu.CompilerParams(needs_layout_passes=False),
    )
    def k(x_hbm, y_hbm, out_x, buf, posv, othv, sem0, sem1):
        wid = lax.axis_index("s") * 2 + lax.axis_index("c")
        bg = lax.rem(wid, 16)
        b0 = bg * BL
        nbase = (wid // 16) * NH
        sems = (sem0, sem1)

        pltpu.sync_copy(x_hbm.at[pl.ds(b0, BL)], posv)
        pltpu.sync_copy(y_hbm.at[pl.ds(b0, BL)], othv)

        zeros16 = jnp.zeros((16,), jnp.float32)

        def zero_half(sl):
            def zbody(i, c):
                nl = i // BL
                bl = i - nl * BL
                for cc in range(OUT_W // 16):
                    buf[sl, nl, bl, pl.ds(cc * 16, 16)] = zeros16
                return c
            lax.fori_loop(0, NL * BL, zbody, 0)

        lanes = lax.iota(jnp.int32, 16)
        lane_hi = lanes >> 3
        lane_bl = lanes & 7
        gvecs = [jnp.full((16,), float(v), jnp.float32) for v in _GVALS]
        zvecs = [zeros16] * KSIZE

        def scatter_chunk(ci, slot, vals):
            nloc0 = ci * NL
            slotv = jnp.full((16,), slot, jnp.int32)

            def gbody(g, c):
                nlv = lane_hi + 2 * g
                nloc = nloc0 + nlv
                nv = nbase + nloc
                inb = nv < N
                p = plsc.load_gather(posv, [lane_bl, nv],
                                     mask=inb).astype(jnp.int32)
                o = plsc.load_gather(othv, [lane_bl, nv],
                                     mask=inb).astype(jnp.int32)
                ul = p - RADIUS
                br = p + RADIUS + 1
                ulo = o - RADIUS
                bro = o + RADIUS + 1
                in_ul = (ul >= 0) & (ul <= OUT_W) & (ulo >= 0) & (ulo <= OUT_W)
                in_br = (br >= 0) & (br <= OUT_W) & (bro >= 0) & (bro <= OUT_W)
                valid = (in_ul | in_br) & (nloc < NH)
                for j in range(KSIZE):
                    col = ul + j
                    m = valid & (col >= 0) & (col < OUT_W)
                    plsc.store_scatter(buf, [slotv, nlv, lane_bl, col],
                                       vals[j], mask=m)
                return c

            lax.fori_loop(0, NL * BL // 16, gbody, 0)

        def issue(ci, slot):
            n0 = nbase + ci * NL
            for sl in range(2):
                @pl.when((slot == sl) & (ci < CHUNKS - 1))
                def _d(sl=sl):
                    pltpu.async_copy(
                        buf.at[sl],
                        out_x.at[pl.ds(n0, NL), pl.ds(b0, BL)],
                        sems[sl])

            @pl.when(ci == CHUNKS - 1)
            def _dl():
                pltpu.async_copy(
                    buf.at[1, pl.ds(0, NLAST)],
                    out_x.at[pl.ds(n0, NLAST), pl.ds(b0, BL)],
                    sem1)

        zero_half(0)
        scatter_chunk(0, 0, gvecs)
        issue(0, 0)
        zero_half(1)

        def body(ci, c):
            slot = lax.rem(ci, 2)

            @pl.when(ci >= 2)
            def _drain_and_restore():
                for sl in range(2):
                    @pl.when(slot == sl)
                    def _w(sl=sl):
                        pltpu.make_async_copy(
                            buf.at[sl],
                            out_x.at[pl.ds(0, NL), pl.ds(0, BL)],
                            sems[sl]).wait()

                scatter_chunk(ci - 2, slot, zvecs)

            scatter_chunk(ci, slot, gvecs)
            issue(ci, slot)
            return c

        lax.fori_loop(1, CHUNKS, body, 0)

        pltpu.make_async_copy(
            buf.at[0], out_x.at[pl.ds(0, NL), pl.ds(0, BL)], sem0).wait()
        pltpu.make_async_copy(
            buf.at[1, pl.ds(0, NLAST)],
            out_x.at[pl.ds(0, NLAST), pl.ds(0, BL)], sem1).wait()

    return k(xpl, ypl)


def _tc_dense_y_body(x_ref, y_ref, o_ref):
    wio = lax.broadcasted_iota(jnp.int32, (8, OUT_W), 1).astype(jnp.float32)
    xi = x_ref[...].astype(jnp.int32)
    yi = y_ref[...].astype(jnp.int32)
    ulx = xi - RADIUS
    uly = yi - RADIUS
    brx = xi + RADIUS + 1
    bry = yi + RADIUS + 1
    in_ul = (ulx >= 0) & (ulx <= OUT_W) & (uly >= 0) & (uly <= OUT_W)
    in_br = (brx >= 0) & (brx <= OUT_W) & (bry >= 0) & (bry <= OUT_W)
    valid = in_ul | in_br
    centers = jnp.where(valid, yi, -30000).astype(jnp.float32)
    for nn in range(N):
        d = wio - centers[:, nn:nn + 1]
        t = d * d
        val = jnp.exp(t * (-1.0 / (2.0 * SIGMA ** 2)))
        o_ref[nn] = jnp.where(t <= float(RADIUS * RADIUS), val, 0.0)


def _tc_dense_y(xpl, ypl, interpret=False):
    return pl.pallas_call(
        _tc_dense_y_body,
        out_shape=jax.ShapeDtypeStruct((N, B, OUT_W), jnp.float32),
        grid=(B // 8,),
        in_specs=[
            pl.BlockSpec((8, N), lambda j: (j, 0)),
            pl.BlockSpec((8, N), lambda j: (j, 0)),
        ],
        out_specs=pl.BlockSpec((N, 8, OUT_W), lambda j: (0, j, 0)),
        interpret=interpret,
    )(xpl, ypl)


def kernel(lmks):
    lm_scaled = lmks * UPSCALE / STRIDE
    xpl = lm_scaled[..., 0]
    ypl = lm_scaled[..., 1]
    ox = _sc_scatter_x(xpl, ypl)
    oy = _tc_dense_y(xpl, ypl)
    return ox.transpose(1, 0, 2), oy.transpose(1, 0, 2)


# speedup vs baseline: 7.2231x; 1.0019x over previous
import functools

import jax
import jax.numpy as jnp
import numpy as np
from jax import lax
from jax.experimental import pallas as pl
from jax.experimental.pallas import tpu as pltpu
from jax.experimental.pallas import tpu_sc as plsc

B, N = 128, 106
OUT_W = 512
UPSCALE = 4
STRIDE = 4
SIGMA = 2.0
RADIUS = int(SIGMA * 3)
KSIZE = 2 * RADIUS + 1

BL = 8
NH = N // 2
NL = 14
CHUNKS = (NH + NL - 1) // NL
NLAST = NH - NL * (CHUNKS - 1)

_GVALS = np.exp(-((np.arange(KSIZE) - RADIUS) ** 2.0) / (2.0 * SIGMA ** 2)).astype(np.float32)


def _sc_scatter_x(xpl, ypl):
    mesh = plsc.VectorSubcoreMesh(core_axis_name="c", subcore_axis_name="s")

    @functools.partial(
        pl.kernel,
        out_type=jax.ShapeDtypeStruct((N, B, OUT_W), jnp.float32),
        mesh=mesh,
        scratch_types=[
            pltpu.VMEM((2, NL, BL, OUT_W), jnp.float32),
            pltpu.VMEM((BL, N), jnp.float32),
            pltpu.VMEM((BL, N), jnp.float32),
            pltpu.SemaphoreType.DMA,
            pltpu.SemaphoreType.DMA,
        ],
        compiler_params=pltpu.CompilerParams(needs_layout_passes=False),
    )
    def k(x_hbm, y_hbm, out_x, buf, posv, othv, sem0, sem1):
        wid = lax.axis_index("s") * 2 + lax.axis_index("c")
        bg = lax.rem(wid, 16)
        b0 = bg * BL
        nbase = (wid // 16) * NH
        sems = (sem0, sem1)

        pltpu.sync_copy(x_hbm.at[pl.ds(b0, BL)], posv)
        pltpu.sync_copy(y_hbm.at[pl.ds(b0, BL)], othv)

        zeros16 = jnp.zeros((16,), jnp.float32)

        def zero_half(sl):
            def zbody(i, c):
                nl = i // BL
                bl = i - nl * BL
                for cc in range(OUT_W // 16):
                    buf[sl, nl, bl, pl.ds(cc * 16, 16)] = zeros16
                return c
            lax.fori_loop(0, NL * BL, zbody, 0)

        lanes = lax.iota(jnp.int32, 16)
        lane_hi = lanes >> 3
        lane_bl = lanes & 7
        gvecs = [jnp.full((16,), float(v), jnp.float32) for v in _GVALS]
        zvecs = [zeros16] * KSIZE

        def scatter_chunk(ci, slot, vals):
            nloc0 = ci * NL
            slotv = jnp.full((16,), slot, jnp.int32)

            def gbody(g, c):
                nlv = lane_hi + 2 * g
                nloc = nloc0 + nlv
                nv = nbase + nloc
                inb = nv < N
                p = plsc.load_gather(posv, [lane_bl, nv],
                                     mask=inb).astype(jnp.int32)
                o = plsc.load_gather(othv, [lane_bl, nv],
                                     mask=inb).astype(jnp.int32)
                ul = p - RADIUS
                br = p + RADIUS + 1
                ulo = o - RADIUS
                bro = o + RADIUS + 1
                in_ul = (ul >= 0) & (ul <= OUT_W) & (ulo >= 0) & (ulo <= OUT_W)
                in_br = (br >= 0) & (br <= OUT_W) & (bro >= 0) & (bro <= OUT_W)
                valid = (in_ul | in_br) & (nloc < NH)
                for j in range(KSIZE):
                    col = ul + j
                    m = valid & (col >= 0) & (col < OUT_W)
                    plsc.store_scatter(buf, [slotv, nlv, lane_bl, col],
                                       vals[j], mask=m)
                return c

            lax.fori_loop(0, NL * BL // 16, gbody, 0)

        def issue(ci, slot):
            n0 = nbase + ci * NL
            for sl in range(2):
                @pl.when((slot == sl) & (ci < CHUNKS - 1))
                def _d(sl=sl):
                    pltpu.async_copy(
                        buf.at[sl],
                        out_x.at[pl.ds(n0, NL), pl.ds(b0, BL)],
                        sems[sl])

            @pl.when(ci == CHUNKS - 1)
            def _dl():
                pltpu.async_copy(
                    buf.at[1, pl.ds(0, NLAST)],
                    out_x.at[pl.ds(n0, NLAST), pl.ds(b0, BL)],
                    sem1)

        zero_half(0)
        scatter_chunk(0, 0, gvecs)
        issue(0, 0)
        zero_half(1)

        def body(ci, c):
            slot = lax.rem(ci, 2)

            @pl.when(ci >= 2)
            def _drain_and_restore():
                for sl in range(2):
                    @pl.when(slot == sl)
                    def _w(sl=sl):
                        pltpu.make_async_copy(
                            buf.at[sl],
                            out_x.at[pl.ds(0, NL), pl.ds(0, BL)],
                            sems[sl]).wait()

                scatter_chunk(ci - 2, slot, zvecs)

            scatter_chunk(ci, slot, gvecs)
            issue(ci, slot)
            return c

        lax.fori_loop(1, CHUNKS, body, 0)

        pltpu.make_async_copy(
            buf.at[0], out_x.at[pl.ds(0, NL), pl.ds(0, BL)], sem0).wait()
        pltpu.make_async_copy(
            buf.at[1, pl.ds(0, NLAST)],
            out_x.at[pl.ds(0, NLAST), pl.ds(0, BL)], sem1).wait()

    return k(xpl, ypl)


def _tc_dense_y_body(x_ref, y_ref, o_ref):
    wio = lax.broadcasted_iota(jnp.int32, (8, OUT_W), 1).astype(jnp.float32)
    xi = x_ref[...].astype(jnp.int32)
    yi = y_ref[...].astype(jnp.int32)
    ulx = xi - RADIUS
    uly = yi - RADIUS
    brx = xi + RADIUS + 1
    bry = yi + RADIUS + 1
    in_ul = (ulx >= 0) & (ulx <= OUT_W) & (uly >= 0) & (uly <= OUT_W)
    in_br = (brx >= 0) & (brx <= OUT_W) & (bry >= 0) & (bry <= OUT_W)
    valid = in_ul | in_br
    centers = jnp.where(valid, yi, -30000).astype(jnp.float32)
    for nn in range(N):
        d = wio - centers[:, nn:nn + 1]
        t = d * d
        val = jnp.exp(t * (-1.0 / (2.0 * SIGMA ** 2)))
        o_ref[nn] = jnp.where(t <= float(RADIUS * RADIUS), val, 0.0)


def _tc_dense_y(xpl, ypl):
    return pl.pallas_call(
        _tc_dense_y_body,
        out_shape=jax.ShapeDtypeStruct((N, B, OUT_W), jnp.float32),
        grid=(B // 8,),
        in_specs=[
            pl.BlockSpec((8, N), lambda j: (j, 0)),
            pl.BlockSpec((8, N), lambda j: (j, 0)),
        ],
        out_specs=pl.BlockSpec((N, 8, OUT_W), lambda j: (0, j, 0)),
    )(xpl, ypl)


def kernel(lmks):
    lm_scaled = lmks * UPSCALE / STRIDE
    xpl = lm_scaled[..., 0]
    ypl = lm_scaled[..., 1]
    ox = _sc_scatter_x(xpl, ypl)
    oy = _tc_dense_y(xpl, ypl)
    return ox.transpose(1, 0, 2), oy.transpose(1, 0, 2)
